# baseline (speedup 1.0000x reference)
import functools

import jax
import jax.numpy as jnp
from jax import lax
from jax.experimental import pallas as pl
from jax.experimental.pallas import tpu as pltpu
from jax.experimental.pallas import tpu_sc as plsc

B, C, H, W = 8, 8, 512, 512
HW = H * W
NC, NS, L = 2, 16, 16
NW = NC * NS
SEG = HW * B // NW
CHUNK = 4096
NCH = SEG // CHUNK
GROUP = 8
NGRP = CHUNK // (GROUP * L)

_mesh = plsc.VectorSubcoreMesh(
    core_axis_name="c", subcore_axis_name="s", num_cores=NC, num_subcores=NS)


@functools.partial(
    pl.kernel,
    out_type=jax.ShapeDtypeStruct((NW, 32, L), jnp.int32),
    mesh=_mesh,
    scratch_types=[
        pltpu.VMEM((2, C, CHUNK), jnp.float32),
        pltpu.VMEM((2, CHUNK), jnp.int32),
        pltpu.VMEM((3 * C, L), jnp.int32),
        pltpu.VMEM((32, L), jnp.int32),
        pltpu.VMEM((L,), jnp.int32),
        pltpu.SemaphoreType.DMA,
        pltpu.SemaphoreType.DMA,
    ],
)
def _dice_sc(logits_hbm, target_hbm, out_hbm, xbuf, tbuf, wacc, res, tmp,
             sem0, sem1):
    cid = lax.axis_index("c")
    sid = lax.axis_index("s")
    wid = sid * NC + cid
    b = wid % B
    wstart = (wid // B) * SEG

    zero = jnp.zeros((L,), jnp.int32)
    one = jnp.full((L,), 1, jnp.int32)
    eightv = jnp.full((L,), C, jnp.int32)

    def fire(g, buf, sem):
        off = wstart + g * CHUNK
        for cls in range(C):
            pltpu.async_copy(
                logits_hbm.at[b, cls, pl.ds(off, CHUNK)],
                xbuf.at[buf, cls], sem)
        pltpu.async_copy(target_hbm.at[b, pl.ds(off, CHUNK)],
                         tbuf.at[buf], sem)

    def drain(buf, sem):
        for cls in range(C):
            pltpu.make_async_copy(
                logits_hbm.at[b, cls, pl.ds(wstart, CHUNK)],
                xbuf.at[buf, cls], sem).wait()
        pltpu.make_async_copy(target_hbm.at[b, pl.ds(wstart, CHUNK)],
                              tbuf.at[buf], sem).wait()

    def compute(buf):
        def group(j, carry):
            acc_t = zero
            acc_i = zero
            acc_p = zero
            for s in range(GROUP):
                idx = (j * GROUP + s) * L
                xs = [xbuf[buf, cls, pl.ds(idx, L)] for cls in range(C)]
                t = tbuf[buf, pl.ds(idx, L)]
                m = xs[0]
                for cls in range(1, C):
                    m = jnp.maximum(m, xs[cls])
                pred = eightv
                for cls in range(C - 1, -1, -1):
                    pred = jnp.where(
                        xs[cls] == m, jnp.full((L,), cls, jnp.int32), pred)
                bit_t = one << (t << 2)
                acc_t = acc_t + bit_t
                acc_i = acc_i + jnp.where(pred == t, bit_t, zero)
                acc_p = acc_p + (one << (pred << 2))
            for cls in range(C):
                sh = 4 * cls
                wacc[cls] = wacc[cls] + ((acc_t >> sh) & 15)
                wacc[C + cls] = wacc[C + cls] + ((acc_i >> sh) & 15)
                wacc[2 * C + cls] = wacc[2 * C + cls] + ((acc_p >> sh) & 15)
            return carry

        lax.fori_loop(0, NGRP, group, 0)

    for r in range(3 * C):
        wacc[r] = zero

    fire(0, 0, sem0)
    fire(1, 1, sem1)

    def outer(i, carry):
        g0 = i * 2

        drain(0, sem0)
        compute(0)

        @pl.when(g0 + 2 < NCH)
        def _():
            fire(g0 + 2, 0, sem0)

        drain(1, sem1)
        compute(1)

        @pl.when(g0 + 3 < NCH)
        def _():
            fire(g0 + 3, 1, sem1)

        return carry

    lax.fori_loop(0, NCH // 2, outer, 0)

    for r in range(3 * C):
        res[r] = wacc[r]
    for r in range(3 * C, 32):
        res[r] = zero
    pltpu.sync_copy(res, out_hbm.at[wid])


def kernel(logits, target):
    parts = _dice_sc(logits.reshape(B, C, HW), target.reshape(B, HW))
    counts = parts[:, :3 * C, :].sum(axis=2).reshape(
        NW // B, B, 3, C).sum(axis=0)
    cnt_t = counts[:, 0, :].astype(jnp.float32)
    cnt_i = counts[:, 1, :].astype(jnp.float32)
    cnt_p = counts[:, 2, :].astype(jnp.float32)
    union = cnt_p + cnt_t - cnt_i
    return jnp.mean(1.0 - (cnt_i + 1.0) / (union + 1.0))


# speedup vs baseline: 2.5962x; 2.5962x over previous
import functools

import jax
import jax.numpy as jnp
from jax import lax
from jax.experimental import pallas as pl
from jax.experimental.pallas import tpu as pltpu
from jax.experimental.pallas import tpu_sc as plsc

B, C, H, W = 8, 8, 512, 512
HW = H * W
NC, NS, L = 2, 16, 16
NW = NC * NS
SEG = HW * B // NW
CHUNK = 4096
NCH = SEG // CHUNK
GROUP = 8
NGRP = CHUNK // (GROUP * L)

_mesh = plsc.VectorSubcoreMesh(
    core_axis_name="c", subcore_axis_name="s", num_cores=NC, num_subcores=NS)


@functools.partial(
    pl.kernel,
    out_type=jax.ShapeDtypeStruct((NW, 32, L), jnp.int32),
    mesh=_mesh,
    scratch_types=[
        pltpu.VMEM((2, C, CHUNK // W, W), jnp.float32),
        pltpu.VMEM((2, CHUNK // W, W), jnp.int32),
        pltpu.VMEM((3 * C, L), jnp.int32),
        pltpu.VMEM((32, L), jnp.int32),
        pltpu.VMEM((L,), jnp.int32),
        pltpu.SemaphoreType.DMA,
        pltpu.SemaphoreType.DMA,
    ],
)
def _dice_sc(logits_hbm, target_hbm, out_hbm, xbuf, tbuf, wacc, res, tmp,
             sem0, sem1):
    cid = lax.axis_index("c")
    sid = lax.axis_index("s")
    wid = sid * NC + cid
    b = wid % B
    wrow = (wid // B) * (SEG // W)
    crows = CHUNK // W

    zero = jnp.zeros((L,), jnp.int32)
    one = jnp.full((L,), 1, jnp.int32)
    eightv = jnp.full((L,), C, jnp.int32)

    def fire(g, buf, sem):
        row = wrow + g * crows
        for cls in range(C):
            pltpu.async_copy(
                logits_hbm.at[b, cls, pl.ds(row, crows)],
                xbuf.at[buf, cls], sem)
        pltpu.async_copy(target_hbm.at[b, pl.ds(row, crows)],
                         tbuf.at[buf], sem)

    def drain(buf, sem):
        for cls in range(C):
            pltpu.make_async_copy(
                logits_hbm.at[b, cls, pl.ds(wrow, crows)],
                xbuf.at[buf, cls], sem).wait()
        pltpu.make_async_copy(target_hbm.at[b, pl.ds(wrow, crows)],
                              tbuf.at[buf], sem).wait()

    vec_per_row = W // L
    grp_per_row = vec_per_row // GROUP

    def compute(buf):
        def group(j, carry):
            r = j // grp_per_row
            q = j % grp_per_row
            acc_t = zero
            acc_i = zero
            acc_p = zero
            for s in range(GROUP):
                idx = (q * GROUP + s) * L
                xs = [xbuf[buf, cls, r, pl.ds(idx, L)] for cls in range(C)]
                t = tbuf[buf, r, pl.ds(idx, L)]
                m = xs[0]
                for cls in range(1, C):
                    m = jnp.maximum(m, xs[cls])
                pred = eightv
                for cls in range(C - 1, -1, -1):
                    pred = jnp.where(
                        xs[cls] == m, jnp.full((L,), cls, jnp.int32), pred)
                bit_t = one << (t << 2)
                acc_t = acc_t + bit_t
                acc_i = acc_i + jnp.where(pred == t, bit_t, zero)
                acc_p = acc_p + (one << (pred << 2))
            for cls in range(C):
                sh = 4 * cls
                wacc[cls] = wacc[cls] + ((acc_t >> sh) & 15)
                wacc[C + cls] = wacc[C + cls] + ((acc_i >> sh) & 15)
                wacc[2 * C + cls] = wacc[2 * C + cls] + ((acc_p >> sh) & 15)
            return carry

        lax.fori_loop(0, NGRP, group, 0)

    for r in range(3 * C):
        wacc[r] = zero

    fire(0, 0, sem0)
    fire(1, 1, sem1)

    def outer(i, carry):
        g0 = i * 2

        drain(0, sem0)
        compute(0)

        @pl.when(g0 + 2 < NCH)
        def _():
            fire(g0 + 2, 0, sem0)

        drain(1, sem1)
        compute(1)

        @pl.when(g0 + 3 < NCH)
        def _():
            fire(g0 + 3, 1, sem1)

        return carry

    lax.fori_loop(0, NCH // 2, outer, 0)

    for r in range(3 * C):
        res[r] = wacc[r]
    for r in range(3 * C, 32):
        res[r] = zero
    pltpu.sync_copy(res, out_hbm.at[wid])


def kernel(logits, target):
    parts = _dice_sc(logits, target)
    counts = parts[:, :3 * C, :].sum(axis=2).reshape(
        NW // B, B, 3, C).sum(axis=0)
    cnt_t = counts[:, 0, :].astype(jnp.float32)
    cnt_i = counts[:, 1, :].astype(jnp.float32)
    cnt_p = counts[:, 2, :].astype(jnp.float32)
    union = cnt_p + cnt_t - cnt_i
    return jnp.mean(1.0 - (cnt_i + 1.0) / (union + 1.0))


# speedup vs baseline: 3.2899x; 1.2672x over previous
import functools

import jax
import jax.numpy as jnp
from jax import lax
from jax.experimental import pallas as pl
from jax.experimental.pallas import tpu as pltpu
from jax.experimental.pallas import tpu_sc as plsc

B, C, H, W = 8, 8, 512, 512
NC, NS, L = 2, 16, 16
NW = NC * NS

SC_ROWS = 256
TC_ROWS = H - SC_ROWS

WROWS = SC_ROWS // (NW // B)
CROWS = 8
CHUNK = CROWS * W
NCH = WROWS // CROWS
GROUP = 8
NGRP = CHUNK // (GROUP * L)

_mesh = plsc.VectorSubcoreMesh(
    core_axis_name="c", subcore_axis_name="s", num_cores=NC, num_subcores=NS)


@functools.partial(
    pl.kernel,
    out_type=jax.ShapeDtypeStruct((NW, 32, L), jnp.int32),
    mesh=_mesh,
    scratch_types=[
        pltpu.VMEM((2, C, CROWS, W), jnp.float32),
        pltpu.VMEM((2, CROWS, W), jnp.int32),
        pltpu.VMEM((3 * C, L), jnp.int32),
        pltpu.VMEM((32, L), jnp.int32),
        pltpu.SemaphoreType.DMA,
        pltpu.SemaphoreType.DMA,
    ],
)
def _dice_sc(logits_hbm, target_hbm, out_hbm, xbuf, tbuf, wacc, res,
             sem0, sem1):
    cid = lax.axis_index("c")
    sid = lax.axis_index("s")
    wid = sid * NC + cid
    b = wid % B
    wrow = TC_ROWS + (wid // B) * WROWS

    zero = jnp.zeros((L,), jnp.int32)
    one = jnp.full((L,), 1, jnp.int32)
    eightv = jnp.full((L,), C, jnp.int32)

    def fire(g, buf, sem):
        row = wrow + g * CROWS
        for cls in range(C):
            pltpu.async_copy(
                logits_hbm.at[b, cls, pl.ds(row, CROWS)],
                xbuf.at[buf, cls], sem)
        pltpu.async_copy(target_hbm.at[b, pl.ds(row, CROWS)],
                         tbuf.at[buf], sem)

    def drain(buf, sem):
        for cls in range(C):
            pltpu.make_async_copy(
                logits_hbm.at[b, cls, pl.ds(wrow, CROWS)],
                xbuf.at[buf, cls], sem).wait()
        pltpu.make_async_copy(target_hbm.at[b, pl.ds(wrow, CROWS)],
                              tbuf.at[buf], sem).wait()

    vec_per_row = W // L
    grp_per_row = vec_per_row // GROUP

    def compute(buf):
        def group(j, carry):
            r = j // grp_per_row
            q = j % grp_per_row
            acc_t = zero
            acc_i = zero
            acc_p = zero
            for s in range(GROUP):
                idx = (q * GROUP + s) * L
                xs = [xbuf[buf, cls, r, pl.ds(idx, L)] for cls in range(C)]
                t = tbuf[buf, r, pl.ds(idx, L)]
                m = xs[0]
                for cls in range(1, C):
                    m = jnp.maximum(m, xs[cls])
                pred = eightv
                for cls in range(C - 1, -1, -1):
                    pred = jnp.where(
                        xs[cls] == m, jnp.full((L,), cls, jnp.int32), pred)
                bit_t = one << (t << 2)
                acc_t = acc_t + bit_t
                acc_i = acc_i + jnp.where(pred == t, bit_t, zero)
                acc_p = acc_p + (one << (pred << 2))
            for cls in range(C):
                sh = 4 * cls
                wacc[cls] = wacc[cls] + ((acc_t >> sh) & 15)
                wacc[C + cls] = wacc[C + cls] + ((acc_i >> sh) & 15)
                wacc[2 * C + cls] = wacc[2 * C + cls] + ((acc_p >> sh) & 15)
            return carry

        lax.fori_loop(0, NGRP, group, 0)

    for r in range(3 * C):
        wacc[r] = zero

    fire(0, 0, sem0)
    fire(1, 1, sem1)

    def outer(i, carry):
        g0 = i * 2

        drain(0, sem0)
        compute(0)

        @pl.when(g0 + 2 < NCH)
        def _():
            fire(g0 + 2, 0, sem0)

        drain(1, sem1)
        compute(1)

        @pl.when(g0 + 3 < NCH)
        def _():
            fire(g0 + 3, 1, sem1)

        return carry

    lax.fori_loop(0, NCH // 2, outer, 0)

    for r in range(3 * C):
        res[r] = wacc[r]
    for r in range(3 * C, 32):
        res[r] = zero
    pltpu.sync_copy(res, out_hbm.at[wid])


RB = 256
NK = TC_ROWS // RB


def _dice_tc_body(logits_ref, target_ref, stats_ref, acc_ref):
    b = pl.program_id(0)
    k = pl.program_id(1)

    @pl.when(k == 0)
    def _init():
        acc_ref[...] = jnp.zeros_like(acc_ref)

    x = logits_ref[0]
    t = target_ref[0]

    m = x[0]
    for c in range(1, C):
        m = jnp.maximum(m, x[c])

    pred = jnp.full(t.shape, C, jnp.int32)
    for c in range(C - 1, -1, -1):
        pred = jnp.where(x[c] == m, c, pred)

    eqpt = pred == t
    for c in range(C):
        pc = pred == c
        tc = t == c
        ic = eqpt & tc
        both = jnp.where(pc, 1.0, 0.0) + jnp.where(tc, 1.0, 0.0)
        acc_ref[pl.ds(c, 1)] += jnp.sum(both, axis=0, keepdims=True)
        acc_ref[pl.ds(C + c, 1)] += jnp.sum(
            jnp.where(ic, 1.0, 0.0), axis=0, keepdims=True)

    @pl.when(k == NK - 1)
    def _flush():
        for c in range(C):
            stats_ref[0, b, c] = jnp.sum(acc_ref[c])
            stats_ref[1, b, c] = jnp.sum(acc_ref[C + c])


def _dice_tc(logits, target):
    return pl.pallas_call(
        _dice_tc_body,
        grid=(B, NK),
        in_specs=[
            pl.BlockSpec((1, C, RB, W), lambda b, k: (b, 0, k, 0)),
            pl.BlockSpec((1, RB, W), lambda b, k: (b, k, 0)),
        ],
        out_specs=pl.BlockSpec(
            (2, B, C), lambda b, k: (0, 0, 0), memory_space=pltpu.SMEM),
        out_shape=jax.ShapeDtypeStruct((2, B, C), jnp.float32),
        scratch_shapes=[
            pltpu.VMEM((2 * C, W), jnp.float32),
        ],
    )(logits, target)


def kernel(logits, target):
    sc_parts = _dice_sc(logits, target)
    tc_stats = _dice_tc(logits, target)

    counts = sc_parts[:, :3 * C, :].sum(axis=2).reshape(
        NW // B, B, 3, C).sum(axis=0).astype(jnp.float32)
    sc_s1 = counts[:, 0, :] + counts[:, 2, :]
    sc_i = counts[:, 1, :]

    s1 = tc_stats[0] + sc_s1
    si = tc_stats[1] + sc_i
    union = s1 - si
    return jnp.mean(1.0 - (si + 1.0) / (union + 1.0))
